# baseline (speedup 1.0000x reference)
import functools

import jax
import jax.numpy as jnp
import numpy as np
from jax import lax
from jax.experimental import pallas as pl
from jax.experimental.pallas import tpu as pltpu
from jax.experimental.pallas import tpu_sc as plsc

_H = _W = 48
_HW = _H * _W
_BC = 12
_L = 16
_NG = _HW // _L
_NC, _NS = 1, 16
_NW = _NC * _NS
_NB = 6
_M0 = np.float32(0.05)


def _build_tables():
    h = np.arange(48, 0, -1, dtype=np.int64)
    w = np.arange(1, 49, dtype=np.int64)
    gx = np.tile(w, 48)
    gy = np.repeat(h, 48)
    dx = gx[:, None] - gx[None, :]
    dy = gy[:, None] - gy[None, :]
    d2i = dx * dx + dy * dy
    knn = np.argsort(d2i, axis=-1, kind="stable").astype(np.int32)
    d2f = np.square(np.sqrt(d2i.astype(np.float32))).astype(np.float32)
    knn_d2 = np.take_along_axis(d2f, knn, axis=-1)
    idx_t = np.ascontiguousarray(knn.reshape(_NG, _L, _HW).transpose(0, 2, 1))
    d2_t = np.ascontiguousarray(knn_d2.reshape(_NG, _L, _HW).transpose(0, 2, 1))
    return idx_t, d2_t


_IDX_T, _D2_T = _build_tables()


def _sqrt16(x):
    i = plsc.bitcast(x, jnp.int32)
    i = jnp.int32(0x5F3759DF) - lax.shift_right_logical(i, 1)
    y = plsc.bitcast(i, jnp.float32)
    for _ in range(3):
        y = y * (jnp.float32(1.5) - jnp.float32(0.5) * x * y * y)
    return x * y


def _bounds_tc_body(w_ref, out_ref):
    total = jnp.sum(w_ref[...], axis=1, keepdims=True)
    out_ref[...] = jnp.broadcast_to(total * _M0, (_BC, _L))


def _bounds_tc(weight):
    return pl.pallas_call(
        _bounds_tc_body,
        out_shape=jax.ShapeDtypeStruct((_BC, _L), jnp.float32),
    )(weight)


def _dtm_body(idx_hbm, d2_hbm, w_hbm, bnd_hbm, out_hbm, w_v, idx_v, d2_v,
              bnd_v, out_v):
    wid = lax.axis_index("s") * _NC + lax.axis_index("c")
    pltpu.sync_copy(w_hbm, w_v)
    pltpu.sync_copy(bnd_hbm, bnd_v)

    def _scan_pass(bcs):
        bcvs = [jnp.full((_L,), bc, jnp.int32) for bc in bcs]
        bndvs = [bnd_v[bc] for bc in bcs]
        nb = len(bcs)

        def cond(s):
            frs = s[1 + 3 * nb:]
            fall = frs[0]
            for f in frs[1:]:
                fall = fall & f
            return jnp.logical_and(s[0] < _HW, jnp.logical_not(jnp.all(fall)))

        def step(s):
            j = s[0]
            cumws = list(s[1:1 + nb])
            cumds = list(s[1 + nb:1 + 2 * nb])
            vals = list(s[1 + 2 * nb:1 + 3 * nb])
            frs = list(s[1 + 3 * nb:])
            idxv = idx_v[j]
            d2v = d2_v[j]
            for i in range(nb):
                wv = plsc.load_gather(w_v, [bcvs[i], idxv])
                cumws[i] = cumws[i] + wv
                cumds[i] = cumds[i] + d2v * wv
                cand = cumds[i] + d2v * (bndvs[i] - cumws[i])
                vals[i] = jnp.where(frs[i], vals[i], cand)
                frs[i] = jnp.logical_or(frs[i], cumws[i] >= bndvs[i])
            return (j + 1, *cumws, *cumds, *vals, *frs)

        z = jnp.zeros((_L,), jnp.float32)
        f0 = jnp.zeros((_L,), jnp.bool_)
        init = (jnp.int32(0), *([z] * (3 * nb)), *([f0] * nb))
        res = lax.while_loop(cond, step, init)
        vals = res[1 + 2 * nb:1 + 3 * nb]
        for i, bc in enumerate(bcs):
            out_v[bc] = _sqrt16(vals[i] / bndvs[i])

    for t in range(-(-_NG // _NW)):
        g = jnp.minimum(wid + _NW * t, _NG - 1)
        pltpu.sync_copy(idx_hbm.at[g], idx_v)
        pltpu.sync_copy(d2_hbm.at[g], d2_v)
        for p in range(_BC // _NB):
            _scan_pass(range(p * _NB, (p + 1) * _NB))
        pltpu.sync_copy(out_v, out_hbm.at[g])


@functools.cache
def _dtm_sc():
    return functools.partial(
        pl.kernel,
        out_type=jax.ShapeDtypeStruct((_NG, _BC, _L), jnp.float32),
        compiler_params=pltpu.CompilerParams(
            needs_layout_passes=False, use_tc_tiling_on_sc=False,
        ),
        mesh=plsc.VectorSubcoreMesh(
            core_axis_name="c", subcore_axis_name="s",
            num_cores=_NC, num_subcores=_NS,
        ),
        scratch_types=[
            pltpu.VMEM((_BC, _HW), jnp.float32),
            pltpu.VMEM((_HW, _L), jnp.int32),
            pltpu.VMEM((_HW, _L), jnp.float32),
            pltpu.VMEM((_BC, _L), jnp.float32),
            pltpu.VMEM((_BC, _L), jnp.float32),
        ],
    )(_dtm_body)


def kernel(input):
    b, c, h, w = input.shape
    weight = input.reshape(_BC, _HW)
    bounds = _bounds_tc(weight)
    dtm = _dtm_sc()(jnp.asarray(_IDX_T), jnp.asarray(_D2_T), weight, bounds)
    return dtm.transpose(1, 0, 2).reshape(b, c, h, w)


# speedup vs baseline: 2.8355x; 2.8355x over previous
import functools

import jax
import jax.numpy as jnp
import numpy as np
from jax import lax
from jax.experimental import pallas as pl
from jax.experimental.pallas import tpu as pltpu
from jax.experimental.pallas import tpu_sc as plsc

_H = _W = 48
_HW = _H * _W
_BC = 12
_L = 16
_NG = _HW // _L
_NP = _HW // 2
_NC, _NS = 2, 16
_NW = _NC * _NS
_NB = 6
_D2MAX = 2 * 47 * 47 + 1
_M0 = np.float32(0.05)


def _build_tables():
    h = np.arange(48, 0, -1, dtype=np.int64)
    w = np.arange(1, 49, dtype=np.int64)
    gx = np.tile(w, 48)
    gy = np.repeat(h, 48)
    dx = gx[:, None] - gx[None, :]
    dy = gy[:, None] - gy[None, :]
    d2i = dx * dx + dy * dy
    knn = np.argsort(d2i, axis=-1, kind="stable").astype(np.int32)
    knn_t = knn.reshape(_NG, _L, _HW).transpose(0, 2, 1)
    packed = np.ascontiguousarray(knn_t[:, 0::2] | (knn_t[:, 1::2] << 16))
    lut = np.square(np.sqrt(np.arange(_D2MAX, dtype=np.float32)))
    return packed, lut.astype(np.float32)


_PACKED_T, _D2_LUT = _build_tables()


def _sqrt16(x):
    i = plsc.bitcast(x, jnp.int32)
    i = jnp.int32(0x5F3759DF) - lax.shift_right_logical(i, 1)
    y = plsc.bitcast(i, jnp.float32)
    for _ in range(3):
        y = y * (jnp.float32(1.5) - jnp.float32(0.5) * x * y * y)
    return x * y


def _bounds_tc_body(w_ref, out_ref):
    total = jnp.sum(w_ref[...], axis=1, keepdims=True)
    out_ref[...] = jnp.broadcast_to(total * _M0, (_BC, _L))


def _bounds_tc(weight):
    return pl.pallas_call(
        _bounds_tc_body,
        out_shape=jax.ShapeDtypeStruct((_BC, _L), jnp.float32),
    )(weight)


def _dtm_body(idx_hbm, lut_hbm, w_hbm, bnd_hbm, out_hbm, w_v, idx_v, lut_v,
              bnd_v, out_v):
    wid = lax.axis_index("s") * _NC + lax.axis_index("c")
    pltpu.sync_copy(w_hbm, w_v)
    pltpu.sync_copy(bnd_hbm, bnd_v)
    pltpu.sync_copy(lut_hbm, lut_v)
    lane = lax.iota(jnp.int32, _L)

    def _scan_pass(bcs, xi, yi):
        bcvs = [jnp.full((_L,), bc, jnp.int32) for bc in bcs]
        bndvs = [bnd_v[bc] for bc in bcs]
        nb = len(bcs)

        def cond(s):
            frs = s[1 + 3 * nb:]
            fall = frs[0]
            for f in frs[1:]:
                fall = fall & f
            return jnp.logical_and(s[0] < _NP, jnp.logical_not(jnp.all(fall)))

        def step(s):
            j = s[0]
            cumws = list(s[1:1 + nb])
            cumds = list(s[1 + nb:1 + 2 * nb])
            vals = list(s[1 + 2 * nb:1 + 3 * nb])
            frs = list(s[1 + 3 * nb:])
            p = idx_v[j]
            for idxv in (
                jnp.bitwise_and(p, jnp.int32(0xFFFF)),
                lax.shift_right_logical(p, 16),
            ):
                xj = lax.rem(idxv, jnp.int32(_W))
                yj = lax.div(idxv, jnp.int32(_W))
                dx = xi - xj
                dy = yi - yj
                d2v = plsc.load_gather(lut_v, [dx * dx + dy * dy])
                for i in range(nb):
                    wv = plsc.load_gather(w_v, [bcvs[i], idxv])
                    cumws[i] = cumws[i] + wv
                    cumds[i] = cumds[i] + d2v * wv
                    cand = cumds[i] + d2v * (bndvs[i] - cumws[i])
                    vals[i] = jnp.where(frs[i], vals[i], cand)
                    frs[i] = jnp.logical_or(frs[i], cumws[i] >= bndvs[i])
            return (j + 1, *cumws, *cumds, *vals, *frs)

        z = jnp.zeros((_L,), jnp.float32)
        f0 = jnp.zeros((_L,), jnp.bool_)
        init = (jnp.int32(0), *([z] * (3 * nb)), *([f0] * nb))
        res = lax.while_loop(cond, step, init)
        vals = res[1 + 2 * nb:1 + 3 * nb]
        for i, bc in enumerate(bcs):
            out_v[bc] = _sqrt16(vals[i] / bndvs[i])

    for t in range(-(-_NG // _NW)):
        g = jnp.minimum(wid + _NW * t, _NG - 1)
        pid = g * _L + lane
        xi = lax.rem(pid, jnp.int32(_W))
        yi = lax.div(pid, jnp.int32(_W))
        pltpu.sync_copy(idx_hbm.at[g], idx_v)
        for p in range(_BC // _NB):
            _scan_pass(range(p * _NB, (p + 1) * _NB), xi, yi)
        pltpu.sync_copy(out_v, out_hbm.at[g])


@functools.cache
def _dtm_sc():
    return functools.partial(
        pl.kernel,
        out_type=jax.ShapeDtypeStruct((_NG, _BC, _L), jnp.float32),
        compiler_params=pltpu.CompilerParams(
            needs_layout_passes=False, use_tc_tiling_on_sc=False,
        ),
        mesh=plsc.VectorSubcoreMesh(
            core_axis_name="c", subcore_axis_name="s",
            num_cores=_NC, num_subcores=_NS,
        ),
        scratch_types=[
            pltpu.VMEM((_BC, _HW), jnp.float32),
            pltpu.VMEM((_NP, _L), jnp.int32),
            pltpu.VMEM((_D2MAX,), jnp.float32),
            pltpu.VMEM((_BC, _L), jnp.float32),
            pltpu.VMEM((_BC, _L), jnp.float32),
        ],
    )(_dtm_body)


def kernel(input):
    b, c, h, w = input.shape
    weight = input.reshape(_BC, _HW)
    bounds = _bounds_tc(weight)
    dtm = _dtm_sc()(
        jnp.asarray(_PACKED_T), jnp.asarray(_D2_LUT), weight, bounds
    )
    return dtm.transpose(1, 0, 2).reshape(b, c, h, w)


# speedup vs baseline: 2.9514x; 1.0409x over previous
import functools

import jax
import jax.numpy as jnp
import numpy as np
from jax import lax
from jax.experimental import pallas as pl
from jax.experimental.pallas import tpu as pltpu
from jax.experimental.pallas import tpu_sc as plsc

_H = _W = 48
_HW = _H * _W
_BC = 12
_L = 16
_NG = _HW // _L
_NP = _HW // 2
_NC, _NS = 2, 16
_NW = _NC * _NS
_NB = 12
_D2MAX = 2 * 47 * 47 + 1
_M0 = np.float32(0.05)


def _build_tables():
    h = np.arange(48, 0, -1, dtype=np.int64)
    w = np.arange(1, 49, dtype=np.int64)
    gx = np.tile(w, 48)
    gy = np.repeat(h, 48)
    dx = gx[:, None] - gx[None, :]
    dy = gy[:, None] - gy[None, :]
    d2i = dx * dx + dy * dy
    knn = np.argsort(d2i, axis=-1, kind="stable").astype(np.int32)
    knn_t = knn.reshape(_NG, _L, _HW).transpose(0, 2, 1)
    packed = np.ascontiguousarray(knn_t[:, 0::2] | (knn_t[:, 1::2] << 16))
    lut = np.square(np.sqrt(np.arange(_D2MAX, dtype=np.float32)))
    return packed, lut.astype(np.float32)


_PACKED_T, _D2_LUT = _build_tables()


def _sqrt16(x):
    i = plsc.bitcast(x, jnp.int32)
    i = jnp.int32(0x5F3759DF) - lax.shift_right_logical(i, 1)
    y = plsc.bitcast(i, jnp.float32)
    for _ in range(3):
        y = y * (jnp.float32(1.5) - jnp.float32(0.5) * x * y * y)
    return x * y


def _bounds_tc_body(w_ref, out_ref):
    total = jnp.sum(w_ref[...], axis=1, keepdims=True)
    out_ref[...] = jnp.broadcast_to(total * _M0, (_BC, _L))


def _bounds_tc(weight):
    return pl.pallas_call(
        _bounds_tc_body,
        out_shape=jax.ShapeDtypeStruct((_BC, _L), jnp.float32),
    )(weight)


def _dtm_body(idx_hbm, lut_hbm, w_hbm, bnd_hbm, out_hbm, w_v, idx_v, lut_v,
              bnd_v, out_v):
    wid = lax.axis_index("s") * _NC + lax.axis_index("c")
    pltpu.sync_copy(w_hbm, w_v)
    pltpu.sync_copy(bnd_hbm, bnd_v)
    pltpu.sync_copy(lut_hbm, lut_v)
    lane = lax.iota(jnp.int32, _L)

    def _scan_pass(bcs, xi, yi):
        bcvs = [jnp.full((_L,), bc, jnp.int32) for bc in bcs]
        bndvs = [bnd_v[bc] for bc in bcs]
        nb = len(bcs)

        def cond(s):
            frs = s[1 + 3 * nb:]
            fall = frs[0]
            for f in frs[1:]:
                fall = fall & f
            return jnp.logical_and(s[0] < _NP, jnp.logical_not(jnp.all(fall)))

        def step(s):
            j = s[0]
            cumws = list(s[1:1 + nb])
            cumds = list(s[1 + nb:1 + 2 * nb])
            vals = list(s[1 + 2 * nb:1 + 3 * nb])
            frs = list(s[1 + 3 * nb:])
            p = idx_v[j]
            for idxv in (
                jnp.bitwise_and(p, jnp.int32(0xFFFF)),
                lax.shift_right_logical(p, 16),
            ):
                xj = lax.rem(idxv, jnp.int32(_W))
                yj = lax.div(idxv, jnp.int32(_W))
                dx = xi - xj
                dy = yi - yj
                d2v = plsc.load_gather(lut_v, [dx * dx + dy * dy])
                for i in range(nb):
                    wv = plsc.load_gather(w_v, [bcvs[i], idxv])
                    cumws[i] = cumws[i] + wv
                    cumds[i] = cumds[i] + d2v * wv
                    cand = cumds[i] + d2v * (bndvs[i] - cumws[i])
                    vals[i] = jnp.where(frs[i], vals[i], cand)
                    frs[i] = jnp.logical_or(frs[i], cumws[i] >= bndvs[i])
            return (j + 1, *cumws, *cumds, *vals, *frs)

        z = jnp.zeros((_L,), jnp.float32)
        f0 = jnp.zeros((_L,), jnp.bool_)
        init = (jnp.int32(0), *([z] * (3 * nb)), *([f0] * nb))
        res = lax.while_loop(cond, step, init)
        vals = res[1 + 2 * nb:1 + 3 * nb]
        for i, bc in enumerate(bcs):
            out_v[bc] = _sqrt16(vals[i] / bndvs[i])

    for t in range(-(-_NG // _NW)):
        g = jnp.minimum(wid + _NW * t, _NG - 1)
        pid = g * _L + lane
        xi = lax.rem(pid, jnp.int32(_W))
        yi = lax.div(pid, jnp.int32(_W))
        pltpu.sync_copy(idx_hbm.at[g], idx_v)
        for p in range(_BC // _NB):
            _scan_pass(range(p * _NB, (p + 1) * _NB), xi, yi)
        pltpu.sync_copy(out_v, out_hbm.at[g])


@functools.cache
def _dtm_sc():
    return functools.partial(
        pl.kernel,
        out_type=jax.ShapeDtypeStruct((_NG, _BC, _L), jnp.float32),
        compiler_params=pltpu.CompilerParams(
            needs_layout_passes=False, use_tc_tiling_on_sc=False,
        ),
        mesh=plsc.VectorSubcoreMesh(
            core_axis_name="c", subcore_axis_name="s",
            num_cores=_NC, num_subcores=_NS,
        ),
        scratch_types=[
            pltpu.VMEM((_BC, _HW), jnp.float32),
            pltpu.VMEM((_NP, _L), jnp.int32),
            pltpu.VMEM((_D2MAX,), jnp.float32),
            pltpu.VMEM((_BC, _L), jnp.float32),
            pltpu.VMEM((_BC, _L), jnp.float32),
        ],
    )(_dtm_body)


def kernel(input):
    b, c, h, w = input.shape
    weight = input.reshape(_BC, _HW)
    bounds = _bounds_tc(weight)
    dtm = _dtm_sc()(
        jnp.asarray(_PACKED_T), jnp.asarray(_D2_LUT), weight, bounds
    )
    return dtm.transpose(1, 0, 2).reshape(b, c, h, w)


# speedup vs baseline: 3.6095x; 1.2230x over previous
import functools

import jax
import jax.numpy as jnp
import numpy as np
from jax import lax
from jax.experimental import pallas as pl
from jax.experimental.pallas import tpu as pltpu
from jax.experimental.pallas import tpu_sc as plsc

_H = _W = 48
_HW = _H * _W
_BC = 12
_L = 16
_NG = _HW // _L
_NP = (_HW // 8) * 3
_NC, _NS = 2, 16
_NW = _NC * _NS
_NB = 6
_D2MAX = 2 * 47 * 47 + 1
_M0 = np.float32(0.05)


def _build_tables():
    h = np.arange(48, 0, -1, dtype=np.int64)
    w = np.arange(1, 49, dtype=np.int64)
    gx = np.tile(w, 48)
    gy = np.repeat(h, 48)
    dx = gx[:, None] - gx[None, :]
    dy = gy[:, None] - gy[None, :]
    d2i = dx * dx + dy * dy
    knn = np.argsort(d2i, axis=-1, kind="stable").astype(np.int64)
    knn_t = knn.reshape(_NG, _L, _HW).transpose(0, 2, 1)
    o = knn_t.reshape(_NG, _HW // 8, 8, _L)
    w0 = o[:, :, 0] | (o[:, :, 1] << 12) | ((o[:, :, 2] & 0xFF) << 24)
    w1 = (o[:, :, 2] >> 8) | (o[:, :, 3] << 4) | (o[:, :, 4] << 16) | \
        ((o[:, :, 5] & 0xF) << 28)
    w2 = (o[:, :, 5] >> 4) | (o[:, :, 6] << 8) | (o[:, :, 7] << 20)
    packed = np.stack([w0, w1, w2], axis=2).reshape(_NG, _NP, _L)
    packed = np.ascontiguousarray(packed.astype(np.uint32).view(np.int32))
    lut = np.square(np.sqrt(np.arange(_D2MAX, dtype=np.float32)))
    return packed, lut.astype(np.float32)


_PACKED_T, _D2_LUT = _build_tables()


def _sqrt16(x):
    i = plsc.bitcast(x, jnp.int32)
    i = jnp.int32(0x5F3759DF) - lax.shift_right_logical(i, 1)
    y = plsc.bitcast(i, jnp.float32)
    for _ in range(3):
        y = y * (jnp.float32(1.5) - jnp.float32(0.5) * x * y * y)
    return x * y


def _bounds_tc_body(w_ref, out_ref):
    total = jnp.sum(w_ref[...], axis=1, keepdims=True)
    out_ref[...] = jnp.broadcast_to(total * _M0, (_BC, _L))


def _bounds_tc(weight):
    return pl.pallas_call(
        _bounds_tc_body,
        out_shape=jax.ShapeDtypeStruct((_BC, _L), jnp.float32),
    )(weight)


def _dtm_body(idx_hbm, lut_hbm, w_hbm, bnd_hbm, out_hbm, w_v, idx_v, lut_v,
              bnd_v, out_v):
    wid = lax.axis_index("s") * _NC + lax.axis_index("c")
    pltpu.sync_copy(w_hbm, w_v)
    pltpu.sync_copy(bnd_hbm, bnd_v)
    pltpu.sync_copy(lut_hbm, lut_v)
    lane = lax.iota(jnp.int32, _L)

    def _scan_pass(bcs, xi, yi):
        bcvs = [jnp.full((_L,), bc, jnp.int32) for bc in bcs]
        bndvs = [bnd_v[bc] for bc in bcs]
        nb = len(bcs)

        def cond(s):
            frs = s[1 + 3 * nb:]
            fall = frs[0]
            for f in frs[1:]:
                fall = fall & f
            return jnp.logical_and(s[0] < _HW // 8, jnp.logical_not(jnp.all(fall)))

        def step(s):
            j = s[0]
            cumws = list(s[1:1 + nb])
            cumds = list(s[1 + nb:1 + 2 * nb])
            vals = list(s[1 + 2 * nb:1 + 3 * nb])
            frs = list(s[1 + 3 * nb:])
            w0 = idx_v[3 * j]
            w1 = idx_v[3 * j + 1]
            w2 = idx_v[3 * j + 2]
            m12 = jnp.int32(0xFFF)
            sr = lax.shift_right_logical
            sl = lax.shift_left
            for idxv in (
                jnp.bitwise_and(w0, m12),
                jnp.bitwise_and(sr(w0, 12), m12),
                jnp.bitwise_or(sr(w0, 24),
                               sl(jnp.bitwise_and(w1, jnp.int32(0xF)), 8)),
                jnp.bitwise_and(sr(w1, 4), m12),
                jnp.bitwise_and(sr(w1, 16), m12),
                jnp.bitwise_or(sr(w1, 28),
                               sl(jnp.bitwise_and(w2, jnp.int32(0xFF)), 4)),
                jnp.bitwise_and(sr(w2, 8), m12),
                sr(w2, 20),
            ):
                xj = lax.rem(idxv, jnp.int32(_W))
                yj = lax.div(idxv, jnp.int32(_W))
                dx = xi - xj
                dy = yi - yj
                d2v = plsc.load_gather(lut_v, [dx * dx + dy * dy])
                for i in range(nb):
                    wv = plsc.load_gather(w_v, [bcvs[i], idxv])
                    cumws[i] = cumws[i] + wv
                    cumds[i] = cumds[i] + d2v * wv
                    cand = cumds[i] + d2v * (bndvs[i] - cumws[i])
                    vals[i] = jnp.where(frs[i], vals[i], cand)
                    frs[i] = jnp.logical_or(frs[i], cumws[i] >= bndvs[i])
            return (j + 1, *cumws, *cumds, *vals, *frs)

        z = jnp.zeros((_L,), jnp.float32)
        f0 = jnp.zeros((_L,), jnp.bool_)
        init = (jnp.int32(0), *([z] * (3 * nb)), *([f0] * nb))
        res = lax.while_loop(cond, step, init)
        vals = res[1 + 2 * nb:1 + 3 * nb]
        for i, bc in enumerate(bcs):
            out_v[bc] = _sqrt16(vals[i] / bndvs[i])

    def _do_group(t, _):
        g = jnp.minimum(wid + _NW * t, _NG - 1)
        pid = g * _L + lane
        xi = lax.rem(pid, jnp.int32(_W))
        yi = lax.div(pid, jnp.int32(_W))
        pltpu.sync_copy(idx_hbm.at[g], idx_v)
        for p in range(_BC // _NB):
            _scan_pass(range(p * _NB, (p + 1) * _NB), xi, yi)
        pltpu.sync_copy(out_v, out_hbm.at[:, pl.ds(g * _L, _L)])
        return 0

    lax.fori_loop(0, -(-_NG // _NW), _do_group, 0)


@functools.cache
def _dtm_sc():
    return functools.partial(
        pl.kernel,
        out_type=jax.ShapeDtypeStruct((_BC, _HW), jnp.float32),
        compiler_params=pltpu.CompilerParams(
            needs_layout_passes=False, use_tc_tiling_on_sc=False,
        ),
        mesh=plsc.VectorSubcoreMesh(
            core_axis_name="c", subcore_axis_name="s",
            num_cores=_NC, num_subcores=_NS,
        ),
        scratch_types=[
            pltpu.VMEM((_BC, _HW), jnp.float32),
            pltpu.VMEM((_NP, _L), jnp.int32),
            pltpu.VMEM((_D2MAX,), jnp.float32),
            pltpu.VMEM((_BC, _L), jnp.float32),
            pltpu.VMEM((_BC, _L), jnp.float32),
        ],
    )(_dtm_body)


def kernel(input):
    b, c, h, w = input.shape
    weight = input.reshape(_BC, _HW)
    bounds = _bounds_tc(weight)
    dtm = _dtm_sc()(
        jnp.asarray(_PACKED_T), jnp.asarray(_D2_LUT), weight, bounds
    )
    return dtm.reshape(b, c, h, w)
